# baseline (speedup 1.0000x reference)
import functools

import jax
import jax.numpy as jnp
from jax import lax
from jax.experimental import pallas as pl
from jax.experimental.pallas import tpu as pltpu
from jax.experimental.pallas import tpu_sc as plsc

_V = 50003
_ES = 32
_DL = 64
_E = 800000

_NC, _NS = 2, 16
_NW = _NC * _NS
_EPW = 25088
_EPAD = _NW * _EPW
_ER = _EPAD // 128
_RPW = _ER // _NW
_VPAD = 50176
_VPT = _VPAD // _NS

_BE = 2048
_NEB = _EPAD // _BE
_BV = 6272
_NVB = _VPAD // _BV

_mesh = plsc.VectorSubcoreMesh(
    core_axis_name="c", subcore_axis_name="s", num_cores=_NC, num_subcores=_NS
)
_sc_params = pltpu.CompilerParams(use_tc_tiling_on_sc=False,
                                  needs_layout_passes=False)


def _wid():
    return lax.axis_index("s") * _NC + lax.axis_index("c")


def _make_gather(table_map, ch, dtype=jnp.bfloat16):
    n_out = len(table_map)
    n_uniq = max(table_map) + 1
    nch = _EPW // ch
    assert nch % 2 == 0 and ch % 8 == 0
    scratch = []
    for _u in range(n_uniq):
        scratch.append(pltpu.VMEM_SHARED((_VPAD, _ES), dtype))
    for _p in range(2):
        for _ in range(n_out):
            scratch.append(pltpu.VMEM((ch,), jnp.int32))
    for _p in range(2):
        for _ in range(n_out):
            scratch.append(pltpu.VMEM((ch, _ES), dtype))
    for _s in range(4):
        scratch.append(pltpu.SemaphoreType.DMA)
    out_type = [jax.ShapeDtypeStruct((_EPAD, _ES), dtype)] * n_out

    @functools.partial(pl.kernel, out_type=out_type, mesh=_mesh,
                       scratch_types=scratch, compiler_params=_sc_params)
    def k(*refs):
        tabs = refs[:n_uniq]
        idxs = refs[n_uniq:n_uniq + n_out]
        outs = refs[n_uniq + n_out:n_uniq + 2 * n_out]
        o = n_uniq + 2 * n_out
        stab = refs[o:o + n_uniq]
        o += n_uniq
        ib = (refs[o:o + n_out], refs[o + n_out:o + 2 * n_out])
        o += 2 * n_out
        rb = (refs[o:o + n_out], refs[o + n_out:o + 2 * n_out])
        o += 2 * n_out
        sg = refs[o:o + 2]
        so = refs[o + 2:o + 4]
        s = lax.axis_index("s")
        w = _wid()

        for u in range(n_uniq):
            pltpu.sync_copy(tabs[u].at[pl.ds(s * _VPT, _VPT)],
                            stab[u].at[pl.ds(s * _VPT, _VPT)])
        plsc.subcore_barrier()

        def load(c, p):
            for t in range(n_out):
                pltpu.sync_copy(idxs[t].at[pl.ds(w * _EPW + c * ch, ch)],
                                ib[p][t])

        def fire(p):
            for t in range(n_out):
                pltpu.make_async_copy(stab[table_map[t]].at[ib[p][t]],
                                      rb[p][t], sg[p]).start()

        def draing(p):
            for t in range(n_out):
                pltpu.make_async_copy(stab[table_map[t]].at[ib[p][t]],
                                      rb[p][t], sg[p]).wait()

        def outw(c, p, start):
            base = w * _EPW + c * ch
            for t in range(n_out):
                d = pltpu.make_async_copy(rb[p][t],
                                          outs[t].at[pl.ds(base, ch)], so[p])
                if start:
                    d.start()
                else:
                    d.wait()

        load(0, 0)
        fire(0)
        load(1, 1)
        fire(1)
        draing(0)
        outw(0, 0, True)

        def body(j, cr):
            ca = 2 * j + 2
            load(ca, 0)
            outw(ca - 2, 0, False)
            fire(0)
            draing(1)
            outw(ca - 1, 1, True)
            cb = 2 * j + 3
            load(cb, 1)
            outw(cb - 2, 1, False)
            fire(1)
            draing(0)
            outw(cb - 1, 0, True)
            return cr

        lax.fori_loop(0, (nch - 2) // 2, body, 0)
        cl = nch - 1
        draing(1)
        outw(cl, 1, True)
        outw(cl - 1, 0, False)
        outw(cl, 1, False)

    return k


_GCH = 224


_gather3 = _make_gather((0, 1, 0), _GCH)
_gather2 = _make_gather((0, 0), _GCH)


@functools.partial(
    pl.kernel,
    out_type=jax.ShapeDtypeStruct((_NW, _VPAD), jnp.float32),
    mesh=_mesh,
    compiler_params=_sc_params,
    scratch_types=[
        pltpu.VMEM((_VPAD,), jnp.float32),
        pltpu.VMEM((_RPW, 128), jnp.int32),
        pltpu.VMEM((_RPW, 128), jnp.int32),
    ],
)
def _present(subj_hbm, obj_hbm, out_hbm, mask, sbuf, obuf):
    w = _wid()
    zeros16 = jnp.zeros((16,), jnp.float32)
    ones16 = jnp.ones((16,), jnp.float32)

    def zbody(i, c):
        off = pl.multiple_of(i * 128, 128)
        for u in range(8):
            mask[pl.ds(off + u * 16, 16)] = zeros16
        return c

    lax.fori_loop(0, _VPAD // 128, zbody, 0)

    pltpu.sync_copy(subj_hbm.at[pl.ds(w * _RPW, _RPW)], sbuf)
    pltpu.sync_copy(obj_hbm.at[pl.ds(w * _RPW, _RPW)], obuf)

    def sbody(r, c):
        for u in range(8):
            iv = sbuf[r, pl.ds(u * 16, 16)]
            plsc.store_scatter(mask, [iv], ones16)
            jv = obuf[r, pl.ds(u * 16, 16)]
            plsc.store_scatter(mask, [jv], ones16)
        return c

    lax.fori_loop(0, _RPW, sbody, 0)
    pltpu.sync_copy(mask, out_hbm.at[w])


_SCH = 256
_SKR = _SCH // 128
_SNCH = _EPW // _SCH


@functools.partial(
    pl.kernel,
    out_type=jax.ShapeDtypeStruct((_NC, _VPAD, _ES), jnp.float32),
    mesh=_mesh,
    compiler_params=_sc_params,
    scratch_types=[
        pltpu.VMEM((_SCH,), jnp.int32),
        pltpu.VMEM((_SCH,), jnp.int32),
        pltpu.VMEM((_SCH, _ES), jnp.float32),
        pltpu.VMEM((_SCH, _ES), jnp.float32),
        pltpu.VMEM((196, _ES), jnp.float32),
        pltpu.VMEM_SHARED((_VPAD, _ES), jnp.float32),
        pltpu.SemaphoreType.DMA,
        pltpu.SemaphoreType.DMA,
    ],
)
def _scatter(msg_hbm, obj_hbm, out_hbm, ib0, ib1, mb0, mb1, zb, shared,
             sem0, sem1):
    c = lax.axis_index("c")
    s = lax.axis_index("s")
    w = _wid()
    ib = (ib0, ib1)
    mb = (mb0, mb1)
    sems = (sem0, sem1)
    zeros16 = jnp.zeros((16,), jnp.float32)

    def zvbody(i, cr):
        zb[i, pl.ds(0, 16)] = zeros16
        zb[i, pl.ds(16, 16)] = zeros16
        return cr

    lax.fori_loop(0, 196, zvbody, 0)

    def zsbody(i, cr):
        pltpu.sync_copy(zb, shared.at[pl.ds(s * _VPT + i * 196, 196)])
        return cr

    lax.fori_loop(0, _VPT // 196, zsbody, 0)
    plsc.subcore_barrier()

    def load(ci, p):
        pltpu.sync_copy(obj_hbm.at[pl.ds(w * _EPW + ci * _SCH, _SCH)], ib[p])
        pltpu.sync_copy(msg_hbm.at[pl.ds(w * _EPW + ci * _SCH, _SCH)], mb[p])

    def fire(ci, p):
        pltpu.async_copy(mb[p], shared.at[ib[p]], sems[p], add=True)

    def drain(p):
        pltpu.make_async_copy(mb[p], shared.at[ib[p]], sems[p]).wait()

    load(0, 0)
    fire(0, 0)
    load(1, 1)
    fire(1, 1)
    drain(0)
    load(2, 0)
    fire(2, 0)

    def body(j, cr):
        ca = 2 * j + 3
        drain(1)
        load(ca, 1)
        fire(ca, 1)
        cb = 2 * j + 4
        drain(0)
        load(cb, 0)
        fire(cb, 0)
        return cr

    lax.fori_loop(0, (_SNCH - 4) // 2, body, 0)
    cl = _SNCH - 1
    drain(1)
    load(cl, 1)
    fire(cl, 1)
    drain(0)
    drain(1)
    plsc.subcore_barrier()

    def fbody(i, cr):
        off = s * _VPT + i * 196
        pltpu.sync_copy(shared.at[pl.ds(off, 196)],
                        out_hbm.at[c, pl.ds(off, 196)])
        return cr

    lax.fori_loop(0, _VPT // 196, fbody, 0)


def _gelu(x):
    return x * 0.5 * (1.0 + lax.erf(x * 0.7071067811865476))


def _mlp1_body(gs, ge, go, w1a, w1b, w1c, b1, w2, b2, out):
    pre = (jnp.dot(gs[...].astype(jnp.float32), w1a[...],
                   preferred_element_type=jnp.float32)
           + jnp.dot(ge[...].astype(jnp.float32), w1b[...],
                     preferred_element_type=jnp.float32)
           + jnp.dot(go[...].astype(jnp.float32), w1c[...],
                     preferred_element_type=jnp.float32)
           + b1[...])
    h = _gelu(pre)
    out[...] = jnp.dot(h, w2[...], preferred_element_type=jnp.float32) + b2[...]


def _mlp1(gs, ge, go, w1a, w1b, w1c, b1, w2, b2):
    eb = pl.BlockSpec((_BE, _ES), lambda i: (i, 0))
    full = lambda shape: pl.BlockSpec(shape, lambda i: tuple(0 for _ in shape))
    return pl.pallas_call(
        _mlp1_body,
        grid=(_NEB,),
        in_specs=[eb, eb, eb, full((_ES, 64)), full((_ES, 64)), full((_ES, 64)),
                  full((1, 64)), full((64, _ES)), full((1, _ES))],
        out_specs=eb,
        out_shape=jax.ShapeDtypeStruct((_EPAD, _ES), jnp.float32),
    )(gs, ge, go, w1a, w1b, w1c, b1, w2, b2)


def _mlp2_body(gs, ge, go, w1a, w1b, w1c, b1, out):
    i = pl.program_id(0)
    pre = (jnp.dot(gs[...].astype(jnp.float32), w1a[...],
                   preferred_element_type=jnp.float32)
           + jnp.dot(ge[...].astype(jnp.float32), w1b[...],
                     preferred_element_type=jnp.float32)
           + jnp.dot(go[...].astype(jnp.float32), w1c[...],
                     preferred_element_type=jnp.float32)
           + b1[...])
    h = _gelu(pre)
    row = i * _BE + lax.broadcasted_iota(jnp.int32, (_BE, 1), 0)
    h = jnp.where(row < _E, h, 0.0)
    part = jnp.sum(h, axis=0, keepdims=True)

    @pl.when(i == 0)
    def _():
        out[...] = jnp.zeros_like(out)

    out[...] += part


def _mlp2(gs, ge, go, w1a, w1b, w1c, b1):
    eb = pl.BlockSpec((_BE, _ES), lambda i: (i, 0))
    full = lambda shape: pl.BlockSpec(shape, lambda i: tuple(0 for _ in shape))
    return pl.pallas_call(
        _mlp2_body,
        grid=(_NEB,),
        in_specs=[eb, eb, eb, full((_ES, 64)), full((_ES, 64)), full((_ES, 64)),
                  full((1, 64))],
        out_specs=pl.BlockSpec((1, 64), lambda i: (0, 0)),
        out_shape=jax.ShapeDtypeStruct((1, 64), jnp.float32),
    )(gs, ge, go, w1a, w1b, w1c, b1)


def _prep2_body(sym, d0, d1, out):
    i = pl.program_id(0)
    row = i * _BV + lax.broadcasted_iota(jnp.int32, (_BV, 1), 0)
    out[...] = jnp.where(row < _V, sym[...] + d0[...] + d1[...],
                         0.0).astype(jnp.bfloat16)


def _prep2(sym, d0, d1):
    vb = pl.BlockSpec((_BV, _ES), lambda i: (i, 0))
    return pl.pallas_call(
        _prep2_body,
        grid=(_NVB,),
        in_specs=[vb, vb, vb],
        out_specs=vb,
        out_shape=jax.ShapeDtypeStruct((_VPAD, _ES), jnp.bfloat16),
    )(sym, d0, d1)


def _final_body(pmask, sym, d0, d1, sumh2, w21, b21, wl, bl, out,
                acc, cnt):
    i = pl.program_id(0)
    row = i * _BV + lax.broadcasted_iota(jnp.int32, (1, _BV), 1)
    rowmask = (row < _V).astype(jnp.float32)
    seen = jnp.max(pmask[...], axis=0, keepdims=True)
    pm = jnp.where(seen > 0.0, 1.0, 0.0) * rowmask
    part = (jnp.dot(pm, sym[...], preferred_element_type=jnp.float32)
            + jnp.dot(rowmask, d0[...] + d1[...],
                      preferred_element_type=jnp.float32))

    @pl.when(i == 0)
    def _():
        acc[...] = jnp.zeros_like(acc)
        cnt[...] = jnp.zeros_like(cnt)

    acc[...] += part
    cnt[...] += jnp.sum(pm).reshape(1, 1)

    @pl.when(i == _NVB - 1)
    def _():
        msg2 = (jnp.dot(sumh2[...], w21[...],
                        preferred_element_type=jnp.float32)
                + float(_E) * b21[...])
        mean = (acc[...] + msg2) / cnt[...]
        out[...] = (jnp.dot(mean, wl[...],
                            preferred_element_type=jnp.float32) + bl[...])


def _final(pmask, sym, d0, d1, sumh2, w21, b21, wl, bl):
    vb = pl.BlockSpec((_BV, _ES), lambda i: (i, 0))
    full = lambda shape: pl.BlockSpec(shape, lambda i: tuple(0 for _ in shape))
    return pl.pallas_call(
        _final_body,
        grid=(_NVB,),
        in_specs=[pl.BlockSpec((_NW, _BV), lambda i: (0, i)), vb, vb, vb,
                  full((1, 64)), full((64, _ES)), full((1, _ES)),
                  full((_ES, _DL)), full((1, _DL))],
        out_specs=pl.BlockSpec((1, _DL), lambda i: (0, 0)),
        out_shape=jax.ShapeDtypeStruct((1, _DL), jnp.float32),
        scratch_shapes=[pltpu.VMEM((1, _ES), jnp.float32),
                        pltpu.VMEM((1, 1), jnp.float32)],
    )(pmask, sym, d0, d1, sumh2, w21, b21, wl, bl)


def kernel(facts, sym_emb, edge_emb, W1_0, b1_0, W2_0, b2_0,
           W1_1, b1_1, W2_1, b2_1, Wl, bl):
    pad_e = _EPAD - _E
    subj = jnp.concatenate([facts[:, 0], jnp.full((pad_e,), _V, jnp.int32)])
    pred = jnp.concatenate([facts[:, 1], jnp.full((pad_e,), _V, jnp.int32)])
    obj = jnp.concatenate([facts[:, 2], jnp.full((pad_e,), _V, jnp.int32)])
    subj2d = subj.reshape(_ER, 128)
    obj2d = obj.reshape(_ER, 128)

    zpad = jnp.zeros((_VPAD - _V, _ES), jnp.float32)
    sym_p = jnp.concatenate([sym_emb, zpad], axis=0)
    edge_p = jnp.concatenate([edge_emb, zpad], axis=0)

    b1_0r = b1_0.reshape(1, 64)
    b2_0r = b2_0.reshape(1, _ES)
    b1_1r = b1_1.reshape(1, 64)
    b2_1r = b2_1.reshape(1, _ES)
    blr = bl.reshape(1, _DL)

    sym_bf = sym_p.astype(jnp.bfloat16)
    edge_bf = edge_p.astype(jnp.bfloat16)
    gs1, ge, go1 = _gather3(sym_bf, edge_bf, subj, pred, obj)
    pmask = _present(subj2d, obj2d)
    msg1 = _mlp1(gs1, ge, go1, W1_0[:_ES], W1_0[_ES:2 * _ES], W1_0[2 * _ES:],
                 b1_0r, W2_0, b2_0r)
    delta = _scatter(msg1, obj)
    nodes2 = _prep2(sym_p, delta[0], delta[1])
    gs2, go2 = _gather2(nodes2, subj, obj)
    sumh2 = _mlp2(gs2, ge, go2, W1_1[:_ES], W1_1[_ES:2 * _ES], W1_1[2 * _ES:],
                  b1_1r)
    return _final(pmask, sym_p, delta[0], delta[1], sumh2, W2_1, b2_1r,
                  Wl, blr)


# speedup vs baseline: 1.5386x; 1.5386x over previous
import functools

import jax
import jax.numpy as jnp
from jax import lax
from jax.experimental import pallas as pl
from jax.experimental.pallas import tpu as pltpu
from jax.experimental.pallas import tpu_sc as plsc

_V = 50003
_ES = 32
_DL = 64
_E = 800000

_NC, _NS = 2, 16
_NW = _NC * _NS
_EPW = 25088
_EPAD = _NW * _EPW
_ER = _EPAD // 128
_RPW = _ER // _NW
_VPAD = 50176
_VPT = _VPAD // _NS

_BE = 2048
_BR = _BE // 4
_NEB = _EPAD // _BE
_BV = 6272
_NVB = _VPAD // _BV

_mesh = plsc.VectorSubcoreMesh(
    core_axis_name="c", subcore_axis_name="s", num_cores=_NC, num_subcores=_NS
)
_sc_params = pltpu.CompilerParams(use_tc_tiling_on_sc=False,
                                  needs_layout_passes=False)


def _wid():
    return lax.axis_index("s") * _NC + lax.axis_index("c")


def _make_gather(table_map, ch, dtype=jnp.bfloat16):
    n_out = len(table_map)
    n_uniq = max(table_map) + 1
    nch = _EPW // ch
    assert nch % 2 == 0 and ch % 8 == 0
    scratch = []
    for _u in range(n_uniq):
        scratch.append(pltpu.VMEM_SHARED((_VPAD, _ES), dtype))
    for _p in range(2):
        for _ in range(n_out):
            scratch.append(pltpu.VMEM((ch,), jnp.int32))
    for _p in range(2):
        for _ in range(n_out):
            scratch.append(pltpu.VMEM((ch, _ES), dtype))
    for _s in range(4):
        scratch.append(pltpu.SemaphoreType.DMA)
    out_type = [jax.ShapeDtypeStruct((_EPAD, _ES), dtype)] * n_out

    @functools.partial(pl.kernel, out_type=out_type, mesh=_mesh,
                       scratch_types=scratch, compiler_params=_sc_params)
    def k(*refs):
        tabs = refs[:n_uniq]
        idxs = refs[n_uniq:n_uniq + n_out]
        outs = refs[n_uniq + n_out:n_uniq + 2 * n_out]
        o = n_uniq + 2 * n_out
        stab = refs[o:o + n_uniq]
        o += n_uniq
        ib = (refs[o:o + n_out], refs[o + n_out:o + 2 * n_out])
        o += 2 * n_out
        rb = (refs[o:o + n_out], refs[o + n_out:o + 2 * n_out])
        o += 2 * n_out
        sg = refs[o:o + 2]
        so = refs[o + 2:o + 4]
        s = lax.axis_index("s")
        w = _wid()

        for u in range(n_uniq):
            pltpu.sync_copy(tabs[u].at[pl.ds(s * _VPT, _VPT)],
                            stab[u].at[pl.ds(s * _VPT, _VPT)])
        plsc.subcore_barrier()

        def load(c, p):
            for t in range(n_out):
                pltpu.sync_copy(idxs[t].at[pl.ds(w * _EPW + c * ch, ch)],
                                ib[p][t])

        def fire(p):
            for t in range(n_out):
                pltpu.make_async_copy(stab[table_map[t]].at[ib[p][t]],
                                      rb[p][t], sg[p]).start()

        def draing(p):
            for t in range(n_out):
                pltpu.make_async_copy(stab[table_map[t]].at[ib[p][t]],
                                      rb[p][t], sg[p]).wait()

        def outw(c, p, start):
            base = w * _EPW + c * ch
            for t in range(n_out):
                d = pltpu.make_async_copy(rb[p][t],
                                          outs[t].at[pl.ds(base, ch)], so[p])
                if start:
                    d.start()
                else:
                    d.wait()

        load(0, 0)
        fire(0)
        load(1, 1)
        fire(1)
        draing(0)
        outw(0, 0, True)

        def body(j, cr):
            ca = 2 * j + 2
            load(ca, 0)
            outw(ca - 2, 0, False)
            fire(0)
            draing(1)
            outw(ca - 1, 1, True)
            cb = 2 * j + 3
            load(cb, 1)
            outw(cb - 2, 1, False)
            fire(1)
            draing(0)
            outw(cb - 1, 0, True)
            return cr

        lax.fori_loop(0, (nch - 2) // 2, body, 0)
        cl = nch - 1
        draing(1)
        outw(cl, 1, True)
        outw(cl - 1, 0, False)
        outw(cl, 1, False)

    return k


_GCH = 224


_gather3 = _make_gather((0, 1, 0), _GCH)
_gather2 = _make_gather((0, 0), _GCH)


@functools.partial(
    pl.kernel,
    out_type=jax.ShapeDtypeStruct((_NW, _VPAD), jnp.float32),
    mesh=_mesh,
    compiler_params=_sc_params,
    scratch_types=[
        pltpu.VMEM((_VPAD,), jnp.float32),
        pltpu.VMEM((_RPW, 128), jnp.int32),
        pltpu.VMEM((_RPW, 128), jnp.int32),
    ],
)
def _present(subj_hbm, obj_hbm, out_hbm, mask, sbuf, obuf):
    w = _wid()
    zeros16 = jnp.zeros((16,), jnp.float32)
    ones16 = jnp.ones((16,), jnp.float32)

    def zbody(i, c):
        off = pl.multiple_of(i * 128, 128)
        for u in range(8):
            mask[pl.ds(off + u * 16, 16)] = zeros16
        return c

    lax.fori_loop(0, _VPAD // 128, zbody, 0)

    pltpu.sync_copy(subj_hbm.at[pl.ds(w * _RPW, _RPW)], sbuf)
    pltpu.sync_copy(obj_hbm.at[pl.ds(w * _RPW, _RPW)], obuf)

    def sbody(r, c):
        for u in range(8):
            iv = sbuf[r, pl.ds(u * 16, 16)]
            plsc.store_scatter(mask, [iv], ones16)
            jv = obuf[r, pl.ds(u * 16, 16)]
            plsc.store_scatter(mask, [jv], ones16)
        return c

    lax.fori_loop(0, _RPW, sbody, 0)
    pltpu.sync_copy(mask, out_hbm.at[w])


_SCH = 256
_SKR = _SCH // 128
_SNCH = _EPW // _SCH


@functools.partial(
    pl.kernel,
    out_type=jax.ShapeDtypeStruct((_NC, _VPAD, _ES), jnp.float32),
    mesh=_mesh,
    compiler_params=_sc_params,
    scratch_types=[
        pltpu.VMEM((_SCH,), jnp.int32),
        pltpu.VMEM((_SCH,), jnp.int32),
        pltpu.VMEM((_SCH, _ES), jnp.float32),
        pltpu.VMEM((_SCH, _ES), jnp.float32),
        pltpu.VMEM((196, _ES), jnp.float32),
        pltpu.VMEM_SHARED((_VPAD, _ES), jnp.float32),
        pltpu.SemaphoreType.DMA,
        pltpu.SemaphoreType.DMA,
    ],
)
def _scatter(msg_hbm, obj_hbm, out_hbm, ib0, ib1, mb0, mb1, zb, shared,
             sem0, sem1):
    c = lax.axis_index("c")
    s = lax.axis_index("s")
    w = _wid()
    ib = (ib0, ib1)
    mb = (mb0, mb1)
    sems = (sem0, sem1)
    zeros16 = jnp.zeros((16,), jnp.float32)

    def zvbody(i, cr):
        zb[i, pl.ds(0, 16)] = zeros16
        zb[i, pl.ds(16, 16)] = zeros16
        return cr

    lax.fori_loop(0, 196, zvbody, 0)

    def zsbody(i, cr):
        pltpu.sync_copy(zb, shared.at[pl.ds(s * _VPT + i * 196, 196)])
        return cr

    lax.fori_loop(0, _VPT // 196, zsbody, 0)
    plsc.subcore_barrier()

    def load(ci, p):
        pltpu.sync_copy(obj_hbm.at[pl.ds(w * _EPW + ci * _SCH, _SCH)], ib[p])
        pltpu.sync_copy(msg_hbm.at[pl.ds(w * _EPW + ci * _SCH, _SCH)], mb[p])

    def fire(ci, p):
        pltpu.async_copy(mb[p], shared.at[ib[p]], sems[p], add=True)

    def drain(p):
        pltpu.make_async_copy(mb[p], shared.at[ib[p]], sems[p]).wait()

    load(0, 0)
    fire(0, 0)
    load(1, 1)
    fire(1, 1)
    drain(0)
    load(2, 0)
    fire(2, 0)

    def body(j, cr):
        ca = 2 * j + 3
        drain(1)
        load(ca, 1)
        fire(ca, 1)
        cb = 2 * j + 4
        drain(0)
        load(cb, 0)
        fire(cb, 0)
        return cr

    lax.fori_loop(0, (_SNCH - 4) // 2, body, 0)
    cl = _SNCH - 1
    drain(1)
    load(cl, 1)
    fire(cl, 1)
    drain(0)
    drain(1)
    plsc.subcore_barrier()

    def fbody(i, cr):
        off = s * _VPT + i * 196
        pltpu.sync_copy(shared.at[pl.ds(off, 196)],
                        out_hbm.at[c, pl.ds(off, 196)])
        return cr

    lax.fori_loop(0, _VPT // 196, fbody, 0)


def _gelu(x):
    return x * 0.5 * (1.0 + lax.erf(x * 0.7071067811865476))


def _mlp1_body(gs, ge, go, w1a, w1b, w1c, b1, w2, b2, out):
    pre = (jnp.dot(gs[...], w1a[...], preferred_element_type=jnp.float32)
           + jnp.dot(ge[...], w1b[...], preferred_element_type=jnp.float32)
           + jnp.dot(go[...], w1c[...], preferred_element_type=jnp.float32)
           + b1[...])
    h = _gelu(pre).astype(jnp.bfloat16)
    out[...] = (jnp.dot(h, w2[...], preferred_element_type=jnp.float32)
                + b2[...])


def _mlp1(gs, ge, go, w1a, w1b, w1c, b1, w2, b2):
    eb = pl.BlockSpec((_BR, 128), lambda i: (i, 0))
    full = lambda shape: pl.BlockSpec(shape, lambda i: tuple(0 for _ in shape))
    return pl.pallas_call(
        _mlp1_body,
        grid=(_NEB,),
        in_specs=[eb, eb, eb, full((128, 256)), full((128, 256)),
                  full((128, 256)), full((1, 256)), full((256, 128)),
                  full((1, 128))],
        out_specs=eb,
        out_shape=jax.ShapeDtypeStruct((_EPAD // 4, 128), jnp.float32),
    )(gs, ge, go, w1a, w1b, w1c, b1, w2, b2)


def _mlp2_body(gs, ge, go, w1a, w1b, w1c, b1, out):
    i = pl.program_id(0)
    pre = (jnp.dot(gs[...], w1a[...], preferred_element_type=jnp.float32)
           + jnp.dot(ge[...], w1b[...], preferred_element_type=jnp.float32)
           + jnp.dot(go[...], w1c[...], preferred_element_type=jnp.float32)
           + b1[...])
    h = _gelu(pre)
    edge = ((i * _BR + lax.broadcasted_iota(jnp.int32, (_BR, 256), 0)) * 4
            + lax.broadcasted_iota(jnp.int32, (_BR, 256), 1) // 64)
    h = jnp.where(edge < _E, h, 0.0)
    part = jnp.sum(h, axis=0, keepdims=True)

    @pl.when(i == 0)
    def _():
        out[...] = jnp.zeros_like(out)

    out[...] += part


def _mlp2(gs, ge, go, w1a, w1b, w1c, b1):
    eb = pl.BlockSpec((_BR, 128), lambda i: (i, 0))
    full = lambda shape: pl.BlockSpec(shape, lambda i: tuple(0 for _ in shape))
    return pl.pallas_call(
        _mlp2_body,
        grid=(_NEB,),
        in_specs=[eb, eb, eb, full((128, 256)), full((128, 256)),
                  full((128, 256)), full((1, 256))],
        out_specs=pl.BlockSpec((1, 256), lambda i: (0, 0)),
        out_shape=jax.ShapeDtypeStruct((1, 256), jnp.float32),
    )(gs, ge, go, w1a, w1b, w1c, b1)


def _prep2_body(sym, d0, d1, out):
    i = pl.program_id(0)
    row = i * _BV + lax.broadcasted_iota(jnp.int32, (_BV, 1), 0)
    out[...] = jnp.where(row < _V, sym[...] + d0[...] + d1[...],
                         0.0).astype(jnp.bfloat16)


def _prep2(sym, d0, d1):
    vb = pl.BlockSpec((_BV, _ES), lambda i: (i, 0))
    return pl.pallas_call(
        _prep2_body,
        grid=(_NVB,),
        in_specs=[vb, vb, vb],
        out_specs=vb,
        out_shape=jax.ShapeDtypeStruct((_VPAD, _ES), jnp.bfloat16),
    )(sym, d0, d1)


def _final_body(pmask, sym, d0, d1, sumh2, w21, b21, wl, bl, out,
                acc, cnt):
    i = pl.program_id(0)
    row = i * _BV + lax.broadcasted_iota(jnp.int32, (1, _BV), 1)
    rowmask = (row < _V).astype(jnp.float32)
    seen = jnp.max(pmask[...], axis=0, keepdims=True)
    pm = jnp.where(seen > 0.0, 1.0, 0.0) * rowmask
    part = (jnp.dot(pm, sym[...], preferred_element_type=jnp.float32)
            + jnp.dot(rowmask, d0[...] + d1[...],
                      preferred_element_type=jnp.float32))

    @pl.when(i == 0)
    def _():
        acc[...] = jnp.zeros_like(acc)
        cnt[...] = jnp.zeros_like(cnt)

    acc[...] += part
    cnt[...] += jnp.sum(pm).reshape(1, 1)

    @pl.when(i == _NVB - 1)
    def _():
        s4 = sumh2[...]
        sh = (s4[:, 0:64] + s4[:, 64:128] + s4[:, 128:192]
              + s4[:, 192:256])
        msg2 = (jnp.dot(sh, w21[...], preferred_element_type=jnp.float32)
                + float(_E) * b21[...])
        mean = (acc[...] + msg2) / cnt[...]
        out[...] = (jnp.dot(mean, wl[...],
                            preferred_element_type=jnp.float32) + bl[...])


def _final(pmask, sym, d0, d1, sumh2, w21, b21, wl, bl):
    vb = pl.BlockSpec((_BV, _ES), lambda i: (i, 0))
    full = lambda shape: pl.BlockSpec(shape, lambda i: tuple(0 for _ in shape))
    return pl.pallas_call(
        _final_body,
        grid=(_NVB,),
        in_specs=[pl.BlockSpec((_NW, _BV), lambda i: (0, i)), vb, vb, vb,
                  full((1, 256)), full((64, _ES)), full((1, _ES)),
                  full((_ES, _DL)), full((1, _DL))],
        out_specs=pl.BlockSpec((1, _DL), lambda i: (0, 0)),
        out_shape=jax.ShapeDtypeStruct((1, _DL), jnp.float32),
        scratch_shapes=[pltpu.VMEM((1, _ES), jnp.float32),
                        pltpu.VMEM((1, 1), jnp.float32)],
    )(pmask, sym, d0, d1, sumh2, w21, b21, wl, bl)


def kernel(facts, sym_emb, edge_emb, W1_0, b1_0, W2_0, b2_0,
           W1_1, b1_1, W2_1, b2_1, Wl, bl):
    pad_e = _EPAD - _E
    subj = jnp.concatenate([facts[:, 0], jnp.full((pad_e,), _V, jnp.int32)])
    pred = jnp.concatenate([facts[:, 1], jnp.full((pad_e,), _V, jnp.int32)])
    obj = jnp.concatenate([facts[:, 2], jnp.full((pad_e,), _V, jnp.int32)])
    subj2d = subj.reshape(_ER, 128)
    obj2d = obj.reshape(_ER, 128)

    zpad = jnp.zeros((_VPAD - _V, _ES), jnp.float32)
    sym_p = jnp.concatenate([sym_emb, zpad], axis=0)
    edge_p = jnp.concatenate([edge_emb, zpad], axis=0)

    b1_0r = b1_0.reshape(1, 64)
    b2_0r = b2_0.reshape(1, _ES)
    b1_1r = b1_1.reshape(1, 64)
    b2_1r = b2_1.reshape(1, _ES)
    blr = bl.reshape(1, _DL)

    sym_bf = sym_p.astype(jnp.bfloat16)
    edge_bf = edge_p.astype(jnp.bfloat16)
    eye4 = jnp.eye(4, dtype=jnp.bfloat16)

    def bd(wm):
        return jnp.kron(eye4, wm.astype(jnp.bfloat16))

    w1a0, w1b0, w1c0 = (bd(W1_0[:_ES]), bd(W1_0[_ES:2 * _ES]),
                        bd(W1_0[2 * _ES:]))
    w1a1, w1b1, w1c1 = (bd(W1_1[:_ES]), bd(W1_1[_ES:2 * _ES]),
                        bd(W1_1[2 * _ES:]))
    w2bd0 = bd(W2_0)
    b1t0 = jnp.tile(b1_0, 4).reshape(1, 256)
    b1t1 = jnp.tile(b1_1, 4).reshape(1, 256)
    b2t0 = jnp.tile(b2_0, 4).reshape(1, 128)

    gs1, ge, go1 = _gather3(sym_bf, edge_bf, subj, pred, obj)
    pmask = _present(subj2d, obj2d)
    r4 = lambda x: x.reshape(_EPAD // 4, 128)
    msg1 = _mlp1(r4(gs1), r4(ge), r4(go1), w1a0, w1b0, w1c0, b1t0,
                 w2bd0, b2t0)
    delta = _scatter(msg1.reshape(_EPAD, _ES), obj)
    nodes2 = _prep2(sym_p, delta[0], delta[1])
    gs2, go2 = _gather2(nodes2, subj, obj)
    sumh2 = _mlp2(r4(gs2), r4(ge), r4(go2), w1a1, w1b1, w1c1, b1t1)
    return _final(pmask, sym_p, delta[0], delta[1], sumh2, W2_1, b2_1r,
                  Wl, blr)
